# Initial kernel scaffold; baseline (speedup 1.0000x reference)
import jax
import jax.numpy as jnp
from jax.experimental import pallas as pl


def kernel(feat, W_agg, weight, nodes, labels, neigh_idx):
    raise NotImplementedError("write your pallas kernel here")


# speedup vs baseline: 5.7526x; 5.7526x over previous
import functools

import jax
import jax.numpy as jnp
from jax import lax
from jax.experimental import pallas as pl
from jax.experimental.pallas import tpu as pltpu
from jax.experimental.pallas import tpu_sc as plsc

NC = 2
NS = 16
CH = 128


def _sc_gather_body(nchunks, k_sample,
                    feat_hbm, nodes_hbm, nidx_hbm, self_out, nsum_out,
                    sidx_v, nidx_v, rows_v, acc_v, sem):
    wid = lax.axis_index("s") * NC + lax.axis_index("c")
    w_base = wid * (nchunks * CH)

    def chunk(c, carry):
        base = w_base + c * CH
        pltpu.sync_copy(nodes_hbm.at[pl.ds(base, CH)], sidx_v)
        pltpu.async_copy(feat_hbm.at[sidx_v], rows_v, sem).wait()
        pltpu.sync_copy(rows_v, self_out.at[pl.ds(base, CH)])
        pltpu.sync_copy(nidx_hbm.at[0, pl.ds(base, CH)], nidx_v)
        pltpu.async_copy(feat_hbm.at[nidx_v], acc_v, sem).wait()
        for k in range(1, k_sample):
            pltpu.sync_copy(nidx_hbm.at[k, pl.ds(base, CH)], nidx_v)
            pltpu.async_copy(feat_hbm.at[nidx_v], acc_v, sem, add=True).wait()
        pltpu.sync_copy(acc_v, nsum_out.at[pl.ds(base, CH)])
        return carry

    lax.fori_loop(0, nchunks, chunk, 0, unroll=False)


def _tc_body(s_ref, n_ref, w1_ref, w2_ref, wt_ref, o_ref, *, inv_k):
    p = jax.lax.Precision.HIGHEST
    h = (jnp.dot(s_ref[...], w1_ref[...], precision=p)
         + jnp.dot(n_ref[...] * inv_k, w2_ref[...], precision=p))
    h = jnp.maximum(h, 0.0)
    o_ref[...] = jnp.dot(h, wt_ref[...], precision=p)


def kernel(feat, W_agg, weight, nodes, labels, neigh_idx):
    del labels
    B = nodes.shape[0]
    K = neigh_idx.shape[1]
    D = feat.shape[1]
    C = weight.shape[0]
    NW = NC * NS
    assert B % (NW * CH) == 0
    nchunks = B // (NW * CH)

    nidx_t = neigh_idx.T

    mesh = plsc.VectorSubcoreMesh(
        core_axis_name="c", subcore_axis_name="s",
        num_cores=NC, num_subcores=NS)
    sc_gather = pl.kernel(
        functools.partial(_sc_gather_body, nchunks, K),
        out_type=(jax.ShapeDtypeStruct((B, D), jnp.float32),
                  jax.ShapeDtypeStruct((B, D), jnp.float32)),
        mesh=mesh,
        scratch_types=[
            pltpu.VMEM((CH,), jnp.int32),
            pltpu.VMEM((CH,), jnp.int32),
            pltpu.VMEM((CH, D), jnp.float32),
            pltpu.VMEM((CH, D), jnp.float32),
            pltpu.SemaphoreType.DMA,
        ],
    )
    self_f, nsum = sc_gather(feat, nodes, nidx_t)

    CP = 8
    w1 = W_agg[:D]
    w2 = W_agg[D:]
    wt = jnp.zeros((D, CP), jnp.float32).at[:, :C].set(weight.T)
    bm = 2048
    out = pl.pallas_call(
        functools.partial(_tc_body, inv_k=1.0 / K),
        grid=(B // bm,),
        in_specs=[
            pl.BlockSpec((bm, D), lambda i: (i, 0)),
            pl.BlockSpec((bm, D), lambda i: (i, 0)),
            pl.BlockSpec((D, D), lambda i: (0, 0)),
            pl.BlockSpec((D, D), lambda i: (0, 0)),
            pl.BlockSpec((D, CP), lambda i: (0, 0)),
        ],
        out_specs=pl.BlockSpec((bm, CP), lambda i: (i, 0)),
        out_shape=jax.ShapeDtypeStruct((B, CP), jnp.float32),
    )(self_f, nsum, w1, w2, wt)
    return out[:, :C]


# speedup vs baseline: 8.9639x; 1.5582x over previous
import functools

import jax
import jax.numpy as jnp
from jax import lax
from jax.experimental import pallas as pl
from jax.experimental.pallas import tpu as pltpu
from jax.experimental.pallas import tpu_sc as plsc

NC = 2
NS = 16
CH = 128


def _sc_gather_body(nchunks, k_sample,
                    feat_hbm, nodes_hbm, nidx_hbm, self_out, nsum_out,
                    sidx_v, nidx_v, rows_v, acc_v, sem_n, sem_s):
    wid = lax.axis_index("s") * NC + lax.axis_index("c")
    w_base = wid * (nchunks * CH)

    def chunk(c, carry):
        base = w_base + c * CH
        pltpu.sync_copy(nidx_hbm.at[pl.ds(0, k_sample), pl.ds(base, CH)],
                        nidx_v)
        pltpu.sync_copy(nodes_hbm.at[pl.ds(base, CH)], sidx_v)
        pltpu.async_copy(feat_hbm.at[nidx_v.at[0]], acc_v, sem_n).wait()
        cp_s = pltpu.async_copy(feat_hbm.at[sidx_v], rows_v, sem_s)
        cps = [pltpu.async_copy(feat_hbm.at[nidx_v.at[k]], acc_v, sem_n,
                                add=True)
               for k in range(1, k_sample)]
        cp_s.wait()
        pltpu.sync_copy(rows_v, self_out.at[pl.ds(base, CH)])
        for cp in cps:
            cp.wait()
        pltpu.sync_copy(acc_v, nsum_out.at[pl.ds(base, CH)])
        return carry

    lax.fori_loop(0, nchunks, chunk, 0, unroll=False)


def _tc_body(s_ref, n_ref, w1_ref, w2_ref, wt_ref, o_ref, *, inv_k):
    p = jax.lax.Precision.HIGHEST
    h = (jnp.dot(s_ref[...], w1_ref[...], precision=p)
         + jnp.dot(n_ref[...] * inv_k, w2_ref[...], precision=p))
    h = jnp.maximum(h, 0.0)
    o_ref[...] = jnp.dot(h, wt_ref[...], precision=p)


def kernel(feat, W_agg, weight, nodes, labels, neigh_idx):
    del labels
    B = nodes.shape[0]
    K = neigh_idx.shape[1]
    D = feat.shape[1]
    C = weight.shape[0]
    NW = NC * NS
    assert B % (NW * CH) == 0
    nchunks = B // (NW * CH)

    nidx_t = neigh_idx.T

    mesh = plsc.VectorSubcoreMesh(
        core_axis_name="c", subcore_axis_name="s",
        num_cores=NC, num_subcores=NS)
    sc_gather = pl.kernel(
        functools.partial(_sc_gather_body, nchunks, K),
        out_type=(jax.ShapeDtypeStruct((B, D), jnp.float32),
                  jax.ShapeDtypeStruct((B, D), jnp.float32)),
        mesh=mesh,
        scratch_types=[
            pltpu.VMEM((CH,), jnp.int32),
            pltpu.VMEM((K, CH), jnp.int32),
            pltpu.VMEM((CH, D), jnp.float32),
            pltpu.VMEM((CH, D), jnp.float32),
            pltpu.SemaphoreType.DMA,
            pltpu.SemaphoreType.DMA,
        ],
    )
    self_f, nsum = sc_gather(feat, nodes, nidx_t)

    CP = 8
    w1 = W_agg[:D]
    w2 = W_agg[D:]
    wt = jnp.zeros((D, CP), jnp.float32).at[:, :C].set(weight.T)
    bm = 2048
    out = pl.pallas_call(
        functools.partial(_tc_body, inv_k=1.0 / K),
        grid=(B // bm,),
        in_specs=[
            pl.BlockSpec((bm, D), lambda i: (i, 0)),
            pl.BlockSpec((bm, D), lambda i: (i, 0)),
            pl.BlockSpec((D, D), lambda i: (0, 0)),
            pl.BlockSpec((D, D), lambda i: (0, 0)),
            pl.BlockSpec((D, CP), lambda i: (0, 0)),
        ],
        out_specs=pl.BlockSpec((bm, CP), lambda i: (i, 0)),
        out_shape=jax.ShapeDtypeStruct((B, CP), jnp.float32),
    )(self_f, nsum, w1, w2, wt)
    return out[:, :C]


# speedup vs baseline: 10.9987x; 1.2270x over previous
import functools

import jax
import jax.numpy as jnp
from jax import lax
from jax.experimental import pallas as pl
from jax.experimental.pallas import tpu as pltpu
from jax.experimental.pallas import tpu_sc as plsc

NC = 2
NS = 16
CH = 128


def _sc_gather_body(nchunks, k_sample,
                    feat_hbm, nodes_hbm, nidx_hbm, self_out, nsum_out,
                    sidx_v, nidx_v, rows_v, acc_v, sem_n, sem_s):
    wid = lax.axis_index("s") * NC + lax.axis_index("c")
    w_base = wid * (nchunks * CH)

    def chunk(c, carry):
        base = w_base + c * CH
        pltpu.sync_copy(nidx_hbm.at[pl.ds(0, k_sample), pl.ds(base, CH)],
                        nidx_v)
        pltpu.sync_copy(nodes_hbm.at[pl.ds(base, CH)], sidx_v)
        pltpu.async_copy(feat_hbm.at[nidx_v.at[0]], acc_v, sem_n).wait()
        cp_s = pltpu.async_copy(feat_hbm.at[sidx_v], rows_v, sem_s)
        cps = [pltpu.async_copy(feat_hbm.at[nidx_v.at[k]], acc_v, sem_n,
                                add=True)
               for k in range(1, k_sample)]
        cp_s.wait()
        pltpu.sync_copy(rows_v, self_out.at[pl.ds(base, CH)])
        for cp in cps:
            cp.wait()
        pltpu.sync_copy(acc_v, nsum_out.at[pl.ds(base, CH)])
        return carry

    lax.fori_loop(0, nchunks, chunk, 0, unroll=False)


def _tc_body(s_ref, n_ref, w1_ref, w2_ref, wt_ref, o_ref, *, inv_k):
    h = (jnp.dot(s_ref[...], w1_ref[...])
         + jnp.dot(n_ref[...] * inv_k, w2_ref[...]))
    h = jnp.maximum(h, 0.0)
    o_ref[...] = jnp.dot(h, wt_ref[...])


def kernel(feat, W_agg, weight, nodes, labels, neigh_idx):
    del labels
    B = nodes.shape[0]
    K = neigh_idx.shape[1]
    D = feat.shape[1]
    C = weight.shape[0]
    NW = NC * NS
    assert B % (NW * CH) == 0
    nchunks = B // (NW * CH)

    nidx_t = neigh_idx.T

    mesh = plsc.VectorSubcoreMesh(
        core_axis_name="c", subcore_axis_name="s",
        num_cores=NC, num_subcores=NS)
    sc_gather = pl.kernel(
        functools.partial(_sc_gather_body, nchunks, K),
        out_type=(jax.ShapeDtypeStruct((B, D), jnp.float32),
                  jax.ShapeDtypeStruct((B, D), jnp.float32)),
        mesh=mesh,
        scratch_types=[
            pltpu.VMEM((CH,), jnp.int32),
            pltpu.VMEM((K, CH), jnp.int32),
            pltpu.VMEM((CH, D), jnp.float32),
            pltpu.VMEM((CH, D), jnp.float32),
            pltpu.SemaphoreType.DMA,
            pltpu.SemaphoreType.DMA,
        ],
    )
    self_f, nsum = sc_gather(feat, nodes, nidx_t)

    CP = 8
    w1 = W_agg[:D]
    w2 = W_agg[D:]
    wt = jnp.zeros((D, CP), jnp.float32).at[:, :C].set(weight.T)
    bm = 2048
    out = pl.pallas_call(
        functools.partial(_tc_body, inv_k=1.0 / K),
        grid=(B // bm,),
        in_specs=[
            pl.BlockSpec((bm, D), lambda i: (i, 0)),
            pl.BlockSpec((bm, D), lambda i: (i, 0)),
            pl.BlockSpec((D, D), lambda i: (0, 0)),
            pl.BlockSpec((D, D), lambda i: (0, 0)),
            pl.BlockSpec((D, CP), lambda i: (0, 0)),
        ],
        out_specs=pl.BlockSpec((bm, CP), lambda i: (i, 0)),
        out_shape=jax.ShapeDtypeStruct((B, CP), jnp.float32),
    )(self_f, nsum, w1, w2, wt)
    return out[:, :C]


# speedup vs baseline: 11.2190x; 1.0200x over previous
import functools

import jax
import jax.numpy as jnp
from jax import lax
from jax.experimental import pallas as pl
from jax.experimental.pallas import tpu as pltpu
from jax.experimental.pallas import tpu_sc as plsc

NC = 2
NS = 16
CH = 128


def _sc_gather_body(nchunks, k_sample,
                    feat_hbm, nodes_hbm, nidx_hbm, self_out, nsum_out,
                    sidx_v, nidx_v, rows_v, acc_v, sem_n, sem_s):
    wid = lax.axis_index("s") * NC + lax.axis_index("c")
    w_base = wid * (nchunks * CH)

    def chunk(c, carry):
        base = w_base + c * CH
        pltpu.sync_copy(nidx_hbm.at[pl.ds(0, k_sample), pl.ds(base, CH)],
                        nidx_v)
        pltpu.sync_copy(nodes_hbm.at[pl.ds(base, CH)], sidx_v)
        pltpu.async_copy(feat_hbm.at[nidx_v.at[0]], acc_v, sem_n).wait()
        cp_s = pltpu.async_copy(feat_hbm.at[sidx_v], rows_v, sem_s)
        cps = [pltpu.async_copy(feat_hbm.at[nidx_v.at[k]], acc_v, sem_n,
                                add=True)
               for k in range(1, k_sample)]
        cp_s.wait()
        pltpu.sync_copy(rows_v, self_out.at[pl.ds(base, CH)])
        for cp in cps:
            cp.wait()
        pltpu.sync_copy(acc_v, nsum_out.at[pl.ds(base, CH)])
        return carry

    lax.fori_loop(0, nchunks, chunk, 0, unroll=False)


def _tc_body(s_ref, n_ref, w1_ref, w2_ref, wt_ref, o_ref, *, inv_k):
    h = (jnp.dot(s_ref[...], w1_ref[...])
         + jnp.dot(n_ref[...] * inv_k, w2_ref[...]))
    h = jnp.maximum(h, 0.0)
    o_ref[...] = jnp.dot(h, wt_ref[...])


def kernel(feat, W_agg, weight, nodes, labels, neigh_idx):
    del labels
    B = nodes.shape[0]
    K = neigh_idx.shape[1]
    D = feat.shape[1]
    C = weight.shape[0]
    NW = NC * NS
    NSPLIT = 2
    BS = B // NSPLIT
    assert BS % (NW * CH) == 0
    nchunks = BS // (NW * CH)

    nidx_t = neigh_idx.T

    mesh = plsc.VectorSubcoreMesh(
        core_axis_name="c", subcore_axis_name="s",
        num_cores=NC, num_subcores=NS)
    sc_gather = pl.kernel(
        functools.partial(_sc_gather_body, nchunks, K),
        out_type=(jax.ShapeDtypeStruct((BS, D), jnp.float32),
                  jax.ShapeDtypeStruct((BS, D), jnp.float32)),
        mesh=mesh,
        scratch_types=[
            pltpu.VMEM((CH,), jnp.int32),
            pltpu.VMEM((K, CH), jnp.int32),
            pltpu.VMEM((CH, D), jnp.float32),
            pltpu.VMEM((CH, D), jnp.float32),
            pltpu.SemaphoreType.DMA,
            pltpu.SemaphoreType.DMA,
        ],
    )

    CP = 8
    w1 = W_agg[:D]
    w2 = W_agg[D:]
    wt = jnp.zeros((D, CP), jnp.float32).at[:, :C].set(weight.T)
    bm = 2048
    tc_dense = pl.pallas_call(
        functools.partial(_tc_body, inv_k=1.0 / K),
        grid=(BS // bm,),
        in_specs=[
            pl.BlockSpec((bm, D), lambda i: (i, 0)),
            pl.BlockSpec((bm, D), lambda i: (i, 0)),
            pl.BlockSpec((D, D), lambda i: (0, 0)),
            pl.BlockSpec((D, D), lambda i: (0, 0)),
            pl.BlockSpec((D, CP), lambda i: (0, 0)),
        ],
        out_specs=pl.BlockSpec((bm, CP), lambda i: (i, 0)),
        out_shape=jax.ShapeDtypeStruct((BS, CP), jnp.float32),
    )
    outs = []
    for s in range(NSPLIT):
        self_f, nsum = sc_gather(
            feat, lax.slice(nodes, (s * BS,), ((s + 1) * BS,)),
            lax.slice(nidx_t, (0, s * BS), (K, (s + 1) * BS)))
        outs.append(tc_dense(self_f, nsum, w1, w2, wt))
    return jnp.concatenate(outs, axis=0)[:, :C]


# speedup vs baseline: 12.3380x; 1.0997x over previous
import functools

import jax
import jax.numpy as jnp
from jax import lax
from jax.experimental import pallas as pl
from jax.experimental.pallas import tpu as pltpu
from jax.experimental.pallas import tpu_sc as plsc

NC = 2
NS = 16
CH = 128


def _sc_gather_body(nchunks, k_sample,
                    feat_hbm, nodes_hbm, nidx_hbm, self_out, nsum_out,
                    sidx_a, nidx_a, rows_a, acc_a,
                    sidx_b, nidx_b, rows_b, acc_b,
                    sem_ia, sem_ib, sem_na, sem_nb, sem_sa, sem_sb):
    wid = lax.axis_index("s") * NC + lax.axis_index("c")
    w_base = wid * (nchunks * CH)
    bufs = ((sidx_a, nidx_a, rows_a, acc_a, sem_ia, sem_na, sem_sa),
            (sidx_b, nidx_b, rows_b, acc_b, sem_ib, sem_nb, sem_sb))

    def pair(c, carry):
        base0 = w_base + 2 * c * CH
        idx_cps, gather_state = [], []
        for i, (sidx, nidx, rows, acc, sem_i, sem_n, sem_s) in enumerate(bufs):
            base = base0 + i * CH
            idx_cps.append((
                pltpu.async_copy(
                    nidx_hbm.at[pl.ds(0, k_sample), pl.ds(base, CH)],
                    nidx, sem_i),
                pltpu.async_copy(nodes_hbm.at[pl.ds(base, CH)], sidx,
                                 sem_i)))
        for i, (sidx, nidx, rows, acc, sem_i, sem_n, sem_s) in enumerate(bufs):
            for cp in idx_cps[i]:
                cp.wait()
            init = pltpu.async_copy(feat_hbm.at[nidx.at[0]], acc, sem_n)
            cp_s = pltpu.async_copy(feat_hbm.at[sidx], rows, sem_s)
            init.wait()
            adds = [pltpu.async_copy(feat_hbm.at[nidx.at[k]], acc, sem_n,
                                     add=True)
                    for k in range(1, k_sample)]
            gather_state.append((cp_s, adds))
        for i, (sidx, nidx, rows, acc, sem_i, sem_n, sem_s) in enumerate(bufs):
            base = base0 + i * CH
            cp_s, adds = gather_state[i]
            cp_s.wait()
            pltpu.sync_copy(rows, self_out.at[pl.ds(base, CH)])
            for cp in adds:
                cp.wait()
            pltpu.sync_copy(acc, nsum_out.at[pl.ds(base, CH)])
        return carry

    lax.fori_loop(0, nchunks // 2, pair, 0, unroll=False)


def _tc_body(s_ref, n_ref, w1_ref, w2_ref, wt_ref, o_ref, *, inv_k):
    h = (jnp.dot(s_ref[...], w1_ref[...])
         + jnp.dot(n_ref[...] * inv_k, w2_ref[...]))
    h = jnp.maximum(h, 0.0)
    o_ref[...] = jnp.dot(h, wt_ref[...])


def kernel(feat, W_agg, weight, nodes, labels, neigh_idx):
    del labels
    B = nodes.shape[0]
    K = neigh_idx.shape[1]
    D = feat.shape[1]
    C = weight.shape[0]
    NW = NC * NS
    NSPLIT = 2
    BS = B // NSPLIT
    assert BS % (NW * CH) == 0
    nchunks = BS // (NW * CH)

    nidx_t = neigh_idx.T

    mesh = plsc.VectorSubcoreMesh(
        core_axis_name="c", subcore_axis_name="s",
        num_cores=NC, num_subcores=NS)
    sc_gather = pl.kernel(
        functools.partial(_sc_gather_body, nchunks, K),
        out_type=(jax.ShapeDtypeStruct((BS, D), jnp.float32),
                  jax.ShapeDtypeStruct((BS, D), jnp.float32)),
        mesh=mesh,
        scratch_types=(
            [pltpu.VMEM((CH,), jnp.int32),
             pltpu.VMEM((K, CH), jnp.int32),
             pltpu.VMEM((CH, D), jnp.float32),
             pltpu.VMEM((CH, D), jnp.float32)] * 2
            + [pltpu.SemaphoreType.DMA] * 6),
    )

    CP = 8
    w1 = W_agg[:D]
    w2 = W_agg[D:]
    wt = jnp.zeros((D, CP), jnp.float32).at[:, :C].set(weight.T)
    bm = 2048
    tc_dense = pl.pallas_call(
        functools.partial(_tc_body, inv_k=1.0 / K),
        grid=(BS // bm,),
        in_specs=[
            pl.BlockSpec((bm, D), lambda i: (i, 0)),
            pl.BlockSpec((bm, D), lambda i: (i, 0)),
            pl.BlockSpec((D, D), lambda i: (0, 0)),
            pl.BlockSpec((D, D), lambda i: (0, 0)),
            pl.BlockSpec((D, CP), lambda i: (0, 0)),
        ],
        out_specs=pl.BlockSpec((bm, CP), lambda i: (i, 0)),
        out_shape=jax.ShapeDtypeStruct((BS, CP), jnp.float32),
    )
    outs = []
    for s in range(NSPLIT):
        self_f, nsum = sc_gather(
            feat, lax.slice(nodes, (s * BS,), ((s + 1) * BS,)),
            lax.slice(nidx_t, (0, s * BS), (K, (s + 1) * BS)))
        outs.append(tc_dense(self_f, nsum, w1, w2, wt))
    return jnp.concatenate(outs, axis=0)[:, :C]


# speedup vs baseline: 12.5574x; 1.0178x over previous
import functools

import jax
import jax.numpy as jnp
from jax import lax
from jax.experimental import pallas as pl
from jax.experimental.pallas import tpu as pltpu
from jax.experimental.pallas import tpu_sc as plsc

NC = 2
NS = 16
CH = 128


def _sc_gather_body(nchunks, k_sample, s_base,
                    feat_hbm, nodes_hbm, nidx_hbm, self_out, nsum_out,
                    sidx_a, nidxt_a, rows_a, acc_a,
                    sidx_b, nidxt_b, rows_b, acc_b,
                    sem_ia, sem_ib, sem_na, sem_nb, sem_sa, sem_sb):
    wid = lax.axis_index("s") * NC + lax.axis_index("c")
    w_base = wid * (nchunks * CH)
    bufs = ((sidx_a, nidxt_a, rows_a, acc_a, sem_ia, sem_na, sem_sa),
            (sidx_b, nidxt_b, rows_b, acc_b, sem_ib, sem_nb, sem_sb))

    def pair(c, carry):
        base0 = w_base + 2 * c * CH
        idx_cps, gather_state = [], []
        for i, (sidx, nidxt, rows, acc, sem_i, sem_n, sem_s) \
                in enumerate(bufs):
            base = base0 + i * CH
            idx_cps.append((
                pltpu.async_copy(
                    nidx_hbm.at[pl.ds(0, k_sample),
                                pl.ds(s_base + base, CH)],
                    nidxt, sem_i),
                pltpu.async_copy(nodes_hbm.at[pl.ds(s_base + base, CH)],
                                 sidx, sem_i)))
        for i, (sidx, nidxt, rows, acc, sem_i, sem_n, sem_s) \
                in enumerate(bufs):
            for cp in idx_cps[i]:
                cp.wait()
            init = pltpu.async_copy(feat_hbm.at[nidxt.at[0]], acc, sem_n)
            cp_s = pltpu.async_copy(feat_hbm.at[sidx], rows, sem_s)
            init.wait()
            adds = [pltpu.async_copy(feat_hbm.at[nidxt.at[k]], acc,
                                     sem_n, add=True)
                    for k in range(1, k_sample)]
            gather_state.append((cp_s, adds))
        for i, (sidx, nidxt, rows, acc, sem_i, sem_n, sem_s) \
                in enumerate(bufs):
            base = base0 + i * CH
            cp_s, adds = gather_state[i]
            cp_s.wait()
            pltpu.sync_copy(rows, self_out.at[pl.ds(base, CH)])
            for cp in adds:
                cp.wait()
            pltpu.sync_copy(acc, nsum_out.at[pl.ds(base, CH)])
        return carry

    lax.fori_loop(0, nchunks // 2, pair, 0, unroll=False)


def _tc_body(s_ref, n_ref, w1_ref, w2_ref, wt_ref, o_ref, *, inv_k):
    h = (jnp.dot(s_ref[...], w1_ref[...])
         + jnp.dot(n_ref[...] * inv_k, w2_ref[...]))
    h = jnp.maximum(h, 0.0)
    o_ref[...] = jnp.dot(h, wt_ref[...])


def kernel(feat, W_agg, weight, nodes, labels, neigh_idx):
    del labels
    B = nodes.shape[0]
    K = neigh_idx.shape[1]
    D = feat.shape[1]
    C = weight.shape[0]
    NW = NC * NS
    NSPLIT = 2
    BS = B // NSPLIT
    assert BS % (NW * CH) == 0
    nchunks = BS // (NW * CH)

    nidx_t = neigh_idx.T
    mesh = plsc.VectorSubcoreMesh(
        core_axis_name="c", subcore_axis_name="s",
        num_cores=NC, num_subcores=NS)

    def make_sc(s_base):
        return pl.kernel(
            functools.partial(_sc_gather_body, nchunks, K, s_base),
            out_type=(jax.ShapeDtypeStruct((BS, D), jnp.float32),
                      jax.ShapeDtypeStruct((BS, D), jnp.float32)),
            mesh=mesh,
            scratch_types=(
                [pltpu.VMEM((CH,), jnp.int32),
                 pltpu.VMEM((K, CH), jnp.int32),
                 pltpu.VMEM((CH, D), jnp.float32),
                 pltpu.VMEM((CH, D), jnp.float32)] * 2
                + [pltpu.SemaphoreType.DMA] * 6),
        )

    CP = 8
    w1 = W_agg[:D]
    w2 = W_agg[D:]
    wt = jnp.zeros((D, CP), jnp.float32).at[:, :C].set(weight.T)
    bm = 2048
    tc_dense = pl.pallas_call(
        functools.partial(_tc_body, inv_k=1.0 / K),
        grid=(BS // bm,),
        in_specs=[
            pl.BlockSpec((bm, D), lambda i: (i, 0)),
            pl.BlockSpec((bm, D), lambda i: (i, 0)),
            pl.BlockSpec((D, D), lambda i: (0, 0)),
            pl.BlockSpec((D, D), lambda i: (0, 0)),
            pl.BlockSpec((D, CP), lambda i: (0, 0)),
        ],
        out_specs=pl.BlockSpec((bm, CP), lambda i: (i, 0)),
        out_shape=jax.ShapeDtypeStruct((BS, CP), jnp.float32),
    )
    outs = []
    for s in range(NSPLIT):
        self_f, nsum = make_sc(s * BS)(feat, nodes, nidx_t)
        outs.append(tc_dense(self_f, nsum, w1, w2, wt))
    return jnp.concatenate(outs, axis=0)[:, :C]
